# baseline (speedup 1.0000x reference)
import jax
import jax.numpy as jnp
from jax import lax
from jax.experimental import pallas as pl
from jax.experimental.pallas import tpu as pltpu
from jax.experimental.pallas import tpu_sc as plsc

SEQ = 2048
BATCH = 16
D = 64
V = 100000
PV = 2049
L = 16
NC = 2
NS = 16
NW = NC * NS
SCALE = 8.0
VECS = SEQ // L
UNROLL = 4


def _emb_body(x_hbm, emb_hbm, pos_hbm, out_hbm,
              row_v, prow_v,
              idx0, pidx0, idx1, pidx1, acc0, acc1,
              xbuf_sh, posbuf_sh, sem_row, sem_in, sem_out):
    cid = lax.axis_index("c")
    sid = lax.axis_index("s")
    wid = cid * NS + sid

    def row_fetch(d):
        return (pltpu.async_copy(emb_hbm.at[d], row_v, sem_row),
                pltpu.async_copy(pos_hbm.at[d], prow_v, sem_row))

    row1 = row_fetch(wid)

    pltpu.sync_copy(x_hbm.at[sid], idx0.at[pl.ds(0, SEQ)])

    H = VECS // 2

    def scan_body(k, carry):
        ca, cb = carry
        va = idx0[pl.ds(k * L, L)]
        vb = idx0[pl.ds((H + k) * L, L)]
        ma = jnp.minimum(va, 1)
        mb = jnp.minimum(vb, 1)
        csa = plsc.cumsum(ma)
        csb = plsc.cumsum(mb)
        pidx0[pl.ds(k * L, L)] = (csa + ca) * ma
        pidx0[pl.ds((H + k) * L, L)] = (csb + cb) * mb
        return ca + jnp.max(csa), cb + jnp.max(csb)

    tot_a, _ = lax.fori_loop(0, H, scan_body,
                             (jnp.int32(0), jnp.int32(0)))

    def fix_body(k, _):
        vb = idx0[pl.ds((H + k) * L, L)]
        mb = jnp.minimum(vb, 1)
        pidx0[pl.ds((H + k) * L, L)] = (pidx0[pl.ds((H + k) * L, L)]
                                        + tot_a * mb)
        return 0

    lax.fori_loop(0, H, fix_body, 0)
    pltpu.sync_copy(idx0.at[pl.ds(0, SEQ)], xbuf_sh.at[pl.ds(sid * SEQ, SEQ)])
    pltpu.sync_copy(pidx0.at[pl.ds(0, SEQ)],
                    posbuf_sh.at[pl.ds(sid * SEQ, SEQ)])
    plsc.subcore_barrier()

    ibufs = ((idx0, pidx0), (idx1, pidx1))
    abufs = (acc0, acc1)
    PB = 2 * SEQ
    steps = [(r, bp) for r in range(D // NW) for bp in range(BATCH // 2)]

    def prefetch(step, slot):
        _, bp = step
        return (pltpu.async_copy(xbuf_sh.at[pl.ds(bp * PB, PB)],
                                 ibufs[slot][0], sem_in),
                pltpu.async_copy(posbuf_sh.at[pl.ds(bp * PB, PB)],
                                 ibufs[slot][1], sem_in))

    pf = {0: prefetch(steps[0], 0)}
    store_h = {}
    for i, (r, bp) in enumerate(steps):
        p = i % 2
        d = wid + r * NW
        if bp == 0:
            for h in (row1 if r == 0 else row2):
                h.wait()
        for h in pf.pop(i):
            h.wait()
        if i + 1 < len(steps):
            pf[i + 1] = prefetch(steps[i + 1], 1 - p)
        for h in store_h.pop(p, ()):
            h.wait()
        idx_v, pidx_v = ibufs[p]
        acc_v = abufs[p]

        def gat_body(k, _):
            for u in range(UNROLL):
                o = (k * UNROLL + u) * L
                tok = idx_v[pl.ds(o, L)]
                pos = pidx_v[pl.ds(o, L)]
                e = plsc.load_gather(row_v, [tok])
                pe = plsc.load_gather(prow_v, [pos])
                acc_v[pl.ds(o, L)] = e * SCALE + pe
            return 0

        lax.fori_loop(0, 2 * VECS // UNROLL, gat_body, 0)
        store_h[p] = (
            pltpu.async_copy(acc_v.at[pl.ds(0, SEQ)],
                             out_hbm.at[2 * bp, d], sem_out),
            pltpu.async_copy(acc_v.at[pl.ds(SEQ, SEQ)],
                             out_hbm.at[2 * bp + 1, d], sem_out))
        if r == 0 and bp == BATCH // 2 - 1:
            row2 = row_fetch(wid + NW)
    for hs in store_h.values():
        for h in hs:
            h.wait()


def kernel(x, emb_table, pos_table):
    x_t = x.T
    emb_t = emb_table.T
    pos_t = pos_table.T
    mesh = plsc.VectorSubcoreMesh(core_axis_name="c", subcore_axis_name="s")
    out_t = pl.kernel(
        _emb_body,
        out_type=jax.ShapeDtypeStruct((BATCH, D, SEQ), jnp.float32),
        mesh=mesh,
        compiler_params=pltpu.CompilerParams(
            use_tc_tiling_on_sc=True, needs_layout_passes=False),
        scratch_types=[
            pltpu.VMEM((V,), jnp.float32),
            pltpu.VMEM((PV,), jnp.float32),
            pltpu.VMEM((2 * SEQ,), jnp.int32),
            pltpu.VMEM((2 * SEQ,), jnp.int32),
            pltpu.VMEM((2 * SEQ,), jnp.int32),
            pltpu.VMEM((2 * SEQ,), jnp.int32),
            pltpu.VMEM((2 * SEQ,), jnp.float32),
            pltpu.VMEM((2 * SEQ,), jnp.float32),
            pltpu.VMEM_SHARED((BATCH * SEQ,), jnp.int32),
            pltpu.VMEM_SHARED((BATCH * SEQ,), jnp.int32),
            pltpu.SemaphoreType.DMA,
            pltpu.SemaphoreType.DMA,
            pltpu.SemaphoreType.DMA,
        ],
    )(x_t, emb_t, pos_t)
    return out_t.transpose(2, 0, 1)


# speedup vs baseline: 1.0048x; 1.0048x over previous
import jax
import jax.numpy as jnp
from jax import lax
from jax.experimental import pallas as pl
from jax.experimental.pallas import tpu as pltpu
from jax.experimental.pallas import tpu_sc as plsc

SEQ = 2048
BATCH = 16
D = 64
V = 100000
PV = 2049
L = 16
NC = 2
NS = 16
NW = NC * NS
SCALE = 8.0
VECS = SEQ // L
UNROLL = 4


def _emb_body(x_hbm, emb_hbm, pos_hbm, out_hbm,
              row_v, prow_v,
              idx0, pidx0, idx1, pidx1, acc0, acc1,
              xbuf_sh, posbuf_sh, sem_row, sem_in, sem_out):
    cid = lax.axis_index("c")
    sid = lax.axis_index("s")
    wid = cid * NS + sid

    def row_fetch(d):
        return (pltpu.async_copy(emb_hbm.at[d], row_v, sem_row),
                pltpu.async_copy(pos_hbm.at[d], prow_v, sem_row))

    row1 = row_fetch(wid)

    pltpu.sync_copy(x_hbm.at[sid], idx0.at[pl.ds(0, SEQ)])

    def scan_body(k, carry):
        v = idx0[pl.ds(k * L, L)]
        m = jnp.minimum(v, 1)
        cs = plsc.cumsum(m)
        pidx0[pl.ds(k * L, L)] = (cs + carry) * m
        return carry + jnp.max(cs)

    lax.fori_loop(0, VECS, scan_body, jnp.int32(0))
    pltpu.sync_copy(idx0.at[pl.ds(0, SEQ)], xbuf_sh.at[pl.ds(sid * SEQ, SEQ)])
    pltpu.sync_copy(pidx0.at[pl.ds(0, SEQ)],
                    posbuf_sh.at[pl.ds(sid * SEQ, SEQ)])
    plsc.subcore_barrier()

    ibufs = ((idx0, pidx0), (idx1, pidx1))
    abufs = (acc0, acc1)
    PB = 2 * SEQ
    steps = [(r, bp) for r in range(D // NW) for bp in range(BATCH // 2)]

    def prefetch(step, slot):
        _, bp = step
        return (pltpu.async_copy(xbuf_sh.at[pl.ds(bp * PB, PB)],
                                 ibufs[slot][0], sem_in),
                pltpu.async_copy(posbuf_sh.at[pl.ds(bp * PB, PB)],
                                 ibufs[slot][1], sem_in))

    pf = {0: prefetch(steps[0], 0)}
    store_h = {}
    for i, (r, bp) in enumerate(steps):
        p = i % 2
        d = wid + r * NW
        if bp == 0:
            for h in (row1 if r == 0 else row2):
                h.wait()
        for h in pf.pop(i):
            h.wait()
        if i + 1 < len(steps):
            pf[i + 1] = prefetch(steps[i + 1], 1 - p)
        for h in store_h.pop(p, ()):
            h.wait()
        idx_v, pidx_v = ibufs[p]
        acc_v = abufs[p]

        def gat_body(k, _):
            for u in range(UNROLL):
                o = (k * UNROLL + u) * L
                tok = idx_v[pl.ds(o, L)]
                pos = pidx_v[pl.ds(o, L)]
                e = plsc.load_gather(row_v, [tok])
                pe = plsc.load_gather(prow_v, [pos])
                acc_v[pl.ds(o, L)] = e * SCALE + pe
            return 0

        lax.fori_loop(0, 2 * VECS // UNROLL, gat_body, 0)
        store_h[p] = (
            pltpu.async_copy(acc_v.at[pl.ds(0, SEQ)],
                             out_hbm.at[2 * bp, d], sem_out),
            pltpu.async_copy(acc_v.at[pl.ds(SEQ, SEQ)],
                             out_hbm.at[2 * bp + 1, d], sem_out))
        if r == 0 and bp == BATCH // 2 - 1:
            row2 = row_fetch(wid + NW)
    for hs in store_h.values():
        for h in hs:
            h.wait()


def kernel(x, emb_table, pos_table):
    x_t = x.T
    emb_t = emb_table.T
    pos_t = pos_table.T
    mesh = plsc.VectorSubcoreMesh(core_axis_name="c", subcore_axis_name="s")
    out_t = pl.kernel(
        _emb_body,
        out_type=jax.ShapeDtypeStruct((BATCH, D, SEQ), jnp.float32),
        mesh=mesh,
        compiler_params=pltpu.CompilerParams(
            use_tc_tiling_on_sc=True, needs_layout_passes=False),
        scratch_types=[
            pltpu.VMEM((V,), jnp.float32),
            pltpu.VMEM((PV,), jnp.float32),
            pltpu.VMEM((2 * SEQ,), jnp.int32),
            pltpu.VMEM((2 * SEQ,), jnp.int32),
            pltpu.VMEM((2 * SEQ,), jnp.int32),
            pltpu.VMEM((2 * SEQ,), jnp.int32),
            pltpu.VMEM((2 * SEQ,), jnp.float32),
            pltpu.VMEM((2 * SEQ,), jnp.float32),
            pltpu.VMEM_SHARED((BATCH * SEQ,), jnp.int32),
            pltpu.VMEM_SHARED((BATCH * SEQ,), jnp.int32),
            pltpu.SemaphoreType.DMA,
            pltpu.SemaphoreType.DMA,
            pltpu.SemaphoreType.DMA,
        ],
    )(x_t, emb_t, pos_t)
    return out_t.transpose(2, 0, 1)
